# baseline (speedup 1.0000x reference)
import functools

import jax
import jax.numpy as jnp
from jax import lax
from jax.experimental import pallas as pl
from jax.experimental.pallas import tpu as pltpu
from jax.experimental.pallas import tpu_sc as plsc

BATCH = 16384
LATENT = 128
OUT_W = 2 * LATENT + 1
BLK_W = OUT_W

NC, NS, L = 2, 16, 16
NW = NC * NS
ROWS_PER_W = BATCH // NW
CHUNK = 128
N_CHUNKS = ROWS_PER_W // CHUNK

_mesh = plsc.VectorSubcoreMesh(core_axis_name="c", subcore_axis_name="s")


@functools.partial(
    pl.kernel,
    out_type=(
        jax.ShapeDtypeStruct((BATCH,), jnp.float32),
        jax.ShapeDtypeStruct((BATCH, OUT_W), jnp.float32),
    ),
    mesh=_mesh,
    scratch_types=[
        pltpu.VMEM((CHUNK,), jnp.int32),
        pltpu.VMEM((CHUNK,), jnp.int32),
        pltpu.VMEM((CHUNK,), jnp.int32),
        pltpu.VMEM((CHUNK, BLK_W), jnp.float32),
        pltpu.VMEM((CHUNK, LATENT), jnp.float32),
        pltpu.VMEM((L * L,), jnp.float32),
        pltpu.VMEM((CHUNK,), jnp.float32),
        pltpu.SemaphoreType.DMA,
        pltpu.SemaphoreType.DMA,
    ],
    compiler_params=pltpu.CompilerParams(needs_layout_passes=False),
)
def _mf_kernel(uid_hbm, iid_hbm, rat_hbm, ut_hbm, it_hbm,
               rating_out, emb_out,
               idx_u, idx_i, rat_v, outblk, irows, tbuf, dots,
               sem_u, sem_i):
    wid = lax.axis_index("s") * NC + lax.axis_index("c")
    lanes = jnp.arange(L, dtype=jnp.int32)

    for ci in range(N_CHUNKS):
        base = wid * ROWS_PER_W + ci * CHUNK
        pltpu.sync_copy(uid_hbm.at[pl.ds(base, CHUNK)], idx_u)
        pltpu.sync_copy(iid_hbm.at[pl.ds(base, CHUNK)], idx_i)
        pltpu.sync_copy(rat_hbm.at[pl.ds(base, CHUNK)], rat_v)
        cu = pltpu.async_copy(ut_hbm.at[idx_u], outblk.at[:, pl.ds(0, LATENT)],
                              sem_u)
        cv = pltpu.async_copy(it_hbm.at[idx_i], irows, sem_i)
        cu.wait()
        cv.wait()

        def group_body(g, _):
            row0 = g * L
            nr = rat_v[pl.ds(row0, L)].astype(jnp.float32) / 5.0
            for rr in range(L):
                r = row0 + rr
                acc = None
                for k in range(LATENT // L):
                    u = outblk[r, pl.ds(k * L, L)]
                    v = irows[r, pl.ds(k * L, L)]
                    outblk[r, pl.ds(LATENT + k * L, L)] = v
                    acc = u * v if acc is None else acc + u * v
                plsc.store_scatter(tbuf, [lanes * L + rr], acc)
            rows = row0 + lanes
            plsc.store_scatter(outblk, [rows, jnp.full((L,), 2 * LATENT)], nr)
            tot = tbuf[pl.ds(0, L)]
            for k in range(1, L):
                tot = tot + tbuf[pl.ds(k * L, L)]
            dots[pl.ds(row0, L)] = tot
            return 0

        lax.fori_loop(0, CHUNK // L, group_body, 0)

        pltpu.sync_copy(dots, rating_out.at[pl.ds(base, CHUNK)])
        pltpu.sync_copy(outblk, emb_out.at[pl.ds(base, CHUNK), :])


def kernel(inputs, user_table, item_table):
    uid = inputs[:, 0]
    iid = inputs[:, 1]
    rat = inputs[:, 2]
    rating_vec, embedded = _mf_kernel(uid, iid, rat, user_table, item_table)
    return rating_vec.reshape(-1, 1), embedded
